# baseline (speedup 1.0000x reference)
import functools

import numpy as np
import jax
import jax.numpy as jnp
from jax.experimental import pallas as pl
from jax.experimental.pallas import tpu as pltpu

SUP = 1

_INF = np.float32(3.0e38)


def _topk_body(d_ref, idx_ref, dist_scr, *, K, S):
    BR = d_ref.shape[1]
    dist_scr[...] = d_ref[0]
    iota = jax.lax.broadcasted_iota(jnp.int32, (BR, S), 1)
    for k in range(K):
        D = dist_scr[...]
        m = jnp.min(D, axis=1, keepdims=True)
        j = jnp.min(jnp.where(D == m, iota, S), axis=1, keepdims=True)
        idx_ref[0, :, k : k + 1] = j
        if k + 1 < K:
            dist_scr[...] = jnp.where(iota == j, _INF, D)


@functools.partial(jax.jit, static_argnames=("K",))
def _topk_from_dist(dist, K):
    bs, v, S = dist.shape
    BR = min(v, 256)
    KPAD = max(128, ((K + 127) // 128) * 128)
    out = pl.pallas_call(
        functools.partial(_topk_body, K=K, S=S),
        grid=(bs, v // BR),
        in_specs=[
            pl.BlockSpec((1, BR, S), lambda b, i: (b, i, 0)),
        ],
        out_specs=pl.BlockSpec((1, BR, KPAD), lambda b, i: (b, i, 0)),
        out_shape=jax.ShapeDtypeStruct((bs, v, KPAD), jnp.int32),
        scratch_shapes=[pltpu.VMEM((BR, S), jnp.float32)],
    )(dist)
    return out[:, :, :K]


def _knn_dist(verts):
    inner = jnp.einsum('bvd,bwd->bvw', verts, verts)
    quad = jnp.sum(verts * verts, axis=2)
    return -2.0 * inner + quad[:, None, :] + quad[:, :, None]


def _nearest_dist(target, source):
    inner = jnp.einsum('bvd,bwd->bvw', target, source)
    s2 = jnp.sum(source * source, axis=2)
    t2 = jnp.sum(target * target, axis=2)
    return s2[:, None, :] + t2[:, :, None] - 2.0 * inner


def _norm(x, axis):
    n = jnp.linalg.norm(x, axis=axis, keepdims=True)
    return x / jnp.maximum(n, 1e-12)


def _take_rows(tensor, index):
    return jax.vmap(lambda t, i: t[i])(tensor, index)


def _ndn(vertices, nbr_idx):
    nbrs = _take_rows(vertices, nbr_idx)
    d = nbrs - vertices[:, :, None, :]
    return _norm(d, -1)


def _conv_surface(p, ndn_n, kernel_num):
    sdn = _norm(p["directions"], 0)
    theta = jax.nn.relu(ndn_n @ sdn)
    return jnp.max(theta, axis=2)


def _conv_layer(p, ndn_n, nbr_idx_n, fm, out_ch):
    sdn = _norm(p["directions"], 0)
    theta = jax.nn.relu(ndn_n @ sdn)
    fout = fm @ p["weights"] + p["bias"]
    fc = fout[:, :, :out_ch]
    fs = _take_rows(fout[:, :, out_ch:], nbr_idx_n)
    act = jnp.max(theta * fs, axis=2)
    return fc + act


def _bn(p, x):
    m = jnp.mean(x, axis=(0, 1))
    var = jnp.var(x, axis=(0, 1))
    return (x - m) / jnp.sqrt(var + 1e-5) * p["gamma"] + p["beta"]


def _fusion_surface(p, vertices, idx101, ndn, dim):
    fm_l = jax.nn.relu(_bn(p["bn_l"], _conv_surface(p["conv_l"], ndn[:, :, :5], dim)))
    fm_m = jax.nn.relu(_bn(p["bn_m0"], _conv_surface(p["conv_m0"], ndn[:, :, :20], dim)))
    fm_m = jax.nn.relu(_bn(p["bn_m1"], _conv_layer(p["conv_m1"], ndn[:, :, :20], idx101[:, :, :20], fm_m, dim)))
    fm_g = jax.nn.relu(_bn(p["bn_g0"], _conv_surface(p["conv_g0"], ndn, dim)))
    fm_g = jax.nn.relu(_bn(p["bn_g1"], _conv_layer(p["conv_g1"], ndn, idx101, fm_g, dim)))
    fm_g = jax.nn.relu(_bn(p["bn_g2"], _conv_layer(p["conv_g2"], ndn, idx101, fm_g, dim)))
    out = jnp.concatenate([fm_l, fm_m, fm_g], axis=2)
    return jax.nn.relu(out @ p["down_w"] + p["down_b"])


def _fusion(p, vertices, idx101, ndn, feat, dim):
    fm_l = jax.nn.relu(_bn(p["bn_l"], _conv_layer(p["conv_l"], ndn[:, :, :5], idx101[:, :, :5], feat, dim)))
    fm_m = jax.nn.relu(_bn(p["bn_m0"], _conv_layer(p["conv_m0"], ndn[:, :, :20], idx101[:, :, :20], feat, dim)))
    fm_m = jax.nn.relu(_bn(p["bn_m1"], _conv_layer(p["conv_m1"], ndn[:, :, :20], idx101[:, :, :20], fm_m, dim)))
    fm_g = jax.nn.relu(_bn(p["bn_g0"], _conv_layer(p["conv_g0"], ndn, idx101, feat, dim)))
    fm_g = jax.nn.relu(_bn(p["bn_g1"], _conv_layer(p["conv_g1"], ndn, idx101, fm_g, dim)))
    fm_g = jax.nn.relu(_bn(p["bn_g0"], _conv_layer(p["conv_g0"], ndn, idx101, fm_g, dim)))
    out = jnp.concatenate([fm_l, fm_m, fm_g], axis=2)
    return jax.nn.relu(out @ p["down_w"] + p["down_b"])


def _pool(vertices, fm, top_idx, rate, nn, seed):
    bs, v, _ = vertices.shape
    nbr = top_idx[:, :, 1 : nn + 1]
    pooled = jnp.max(_take_rows(fm, nbr), axis=2)
    pool_num = v // rate
    idx = jnp.asarray(np.random.RandomState(seed).permutation(v)[:pool_num])
    return vertices[:, idx, :], pooled[:, idx, :]


def kernel(vertices, onehot, params):
    vertices = jnp.transpose(vertices, (0, 2, 1))
    bs, v, _ = vertices.shape

    top0 = _topk_from_dist(_knn_dist(vertices), 101)
    idx101_0 = top0[:, :, 1:]
    ndn0 = _ndn(vertices, idx101_0)

    fm_0 = _fusion_surface(params["conv_0"], vertices, idx101_0, ndn0, 32)
    return fm_0
    fm_1 = _fusion(params["conv_1"], vertices, idx101_0, ndn0, fm_0, 64)
    v1, fp1 = _pool(vertices, fm_1, top0, 4, 4, 1)

    top1 = _topk_from_dist(_knn_dist(v1), 101)
    idx101_1 = top1[:, :, 1:]
    ndn1 = _ndn(v1, idx101_1)

    fm_2 = _fusion(params["conv_2"], v1, idx101_1, ndn1, fp1, 128)
    fm_3 = _fusion(params["conv_3"], v1, idx101_1, ndn1, fm_2, 256)
    v2, fp2 = _pool(v1, fm_3, top1, 4, 4, 2)

    top2 = _topk_from_dist(_knn_dist(v2), 101)
    idx101_2 = top2[:, :, 1:]
    ndn2 = _ndn(v2, idx101_2)

    fm_4 = _fusion(params["conv_4"], v2, idx101_2, ndn2, fp2, 512)
    f_global = jnp.max(fm_4, axis=1)

    n1 = _topk_from_dist(_nearest_dist(vertices, v1), 1)
    n2 = _topk_from_dist(_nearest_dist(vertices, v2), 1)
    fm_2u = _take_rows(fm_2, n1)[:, :, 0, :]
    fm_3u = _take_rows(fm_3, n1)[:, :, 0, :]
    fm_4u = _take_rows(fm_4, n2)[:, :, 0, :]

    fg = jnp.broadcast_to(f_global[:, None, :], (bs, v, f_global.shape[-1]))
    oh = jnp.broadcast_to(onehot[:, None, :], (bs, v, onehot.shape[-1]))
    fuse = jnp.concatenate([fm_0, fm_1, fm_2u, fm_3u, fm_4u, fg, oh], axis=2)
    x = jax.nn.relu(fuse @ params["c1_w"] + params["c1_b"])
    x = jax.nn.relu(x @ params["c2_w"] + params["c2_b"])
    x = x @ params["c3_w"] + params["c3_b"]
    return jax.nn.log_softmax(x, axis=-1)


# speedup vs baseline: 4.6366x; 4.6366x over previous
import functools

import numpy as np
import jax
import jax.numpy as jnp
from jax.experimental import pallas as pl
from jax.experimental.pallas import tpu as pltpu
from jax.experimental.pallas import tpu_sc as plsc

_INF = np.float32(3.0e38)


def _topk_body(d_ref, idx_ref, dist_scr, *, K, S):
    BR = d_ref.shape[1]
    dist_scr[...] = d_ref[0]
    iota = jax.lax.broadcasted_iota(jnp.int32, (BR, S), 1)
    for k in range(K):
        D = dist_scr[...]
        m = jnp.min(D, axis=1, keepdims=True)
        j = jnp.min(jnp.where(D == m, iota, S), axis=1, keepdims=True)
        idx_ref[0, :, k : k + 1] = j
        if k + 1 < K:
            dist_scr[...] = jnp.where(iota == j, _INF, D)


@functools.partial(jax.jit, static_argnames=("K",))
def _topk_from_dist(dist, K):
    bs, v, S = dist.shape
    BR = min(v, 256)
    KPAD = max(128, ((K + 127) // 128) * 128)
    out = pl.pallas_call(
        functools.partial(_topk_body, K=K, S=S),
        grid=(bs, v // BR),
        in_specs=[pl.BlockSpec((1, BR, S), lambda b, i: (b, i, 0))],
        out_specs=pl.BlockSpec((1, BR, KPAD), lambda b, i: (b, i, 0)),
        out_shape=jax.ShapeDtypeStruct((bs, v, KPAD), jnp.int32),
        scratch_shapes=[pltpu.VMEM((BR, S), jnp.float32)],
    )(dist)
    return out[:, :, :K]


def _knn_dist(verts):
    inner = jnp.einsum('bvd,bwd->bvw', verts, verts)
    quad = jnp.sum(verts * verts, axis=2)
    return -2.0 * inner + quad[:, None, :] + quad[:, :, None]


def _nearest_dist(target, source):
    inner = jnp.einsum('bvd,bwd->bvw', target, source)
    s2 = jnp.sum(source * source, axis=2)
    t2 = jnp.sum(target * target, axis=2)
    return s2[:, None, :] + t2[:, :, None] - 2.0 * inner


_SC_MESH = None


def _sc_mesh():
    global _SC_MESH
    if _SC_MESH is None:
        _SC_MESH = plsc.VectorSubcoreMesh(core_axis_name="c", subcore_axis_name="s")
    return _SC_MESH


def _sc_gather(table, idx):
    N = idx.shape[0]
    C = table.shape[1]
    assert C % 128 == 0 and C <= 256, C
    W = 128
    assert N % W == 0, (N, W)
    idx2 = idx.reshape(1, N)

    @functools.partial(
        pl.kernel,
        out_type=jax.ShapeDtypeStruct((N, C), table.dtype),
        mesh=_sc_mesh(),
    )
    def k(tab_hbm, i_hbm, o_hbm):
        def body(i_vmem, o_vmem):
            pltpu.sync_copy(tab_hbm.at[i_vmem.at[0]], o_vmem)

        pltpu.emit_pipeline(
            body,
            grid=(N // W,),
            in_specs=[pl.BlockSpec((1, W), lambda i: (0, i))],
            out_specs=[pl.BlockSpec((W, C), lambda i: (i, 0))],
            core_axis_name=("c", "s"),
            dimension_semantics=(pltpu.PARALLEL,),
        )(i_hbm, o_hbm)

    return k(table, idx2)


def _flat_idx(idx, rows_per_batch):
    bs = idx.shape[0]
    off = (jnp.arange(bs, dtype=jnp.int32) * rows_per_batch).reshape(
        (bs,) + (1,) * (idx.ndim - 1)
    )
    return (idx + off).reshape(-1)


def _gather_nd(table_b, idx):
    bs, R, C = table_b.shape
    CP = ((C + 127) // 128) * 128
    if CP != C:
        table_b = jnp.pad(table_b, ((0, 0), (0, 0), (0, CP - C)))
    flat = _flat_idx(idx, R)
    tab = table_b.reshape(bs * R, CP)
    if CP <= 256:
        out = _sc_gather(tab, flat)
    else:
        chunks = [
            _sc_gather(tab[:, c0 : c0 + 256], flat) for c0 in range(0, CP, 256)
        ]
        out = jnp.concatenate(chunks, axis=1)
    return out.reshape(idx.shape + (CP,))


def _ndn_body(nx_ref, c_ref, o_ref):
    nx = nx_ref[0][:, :, :16]
    c = c_ref[0][:, None, :]
    d = nx - c
    nrm = jnp.sqrt(jnp.sum(d * d, axis=2, keepdims=True))
    o_ref[0] = d / jnp.maximum(nrm, 1e-12)


def _ndn16(nxyz, verts16):
    bs, v, n, cp = nxyz.shape
    BV = min(v, 64)
    return pl.pallas_call(
        _ndn_body,
        grid=(bs, v // BV),
        in_specs=[
            pl.BlockSpec((1, BV, n, cp), lambda b, i: (b, i, 0, 0)),
            pl.BlockSpec((1, BV, 16), lambda b, i: (b, i, 0)),
        ],
        out_specs=pl.BlockSpec((1, BV, n, 16), lambda b, i: (b, i, 0, 0)),
        out_shape=jax.ShapeDtypeStruct((bs, v, n, 16), jnp.float32),
    )(nxyz, verts16)


def _b16(x):
    return x.astype(jnp.bfloat16).astype(jnp.float32)


def _theta(nd, s_ref):
    dx = _b16(nd[:, :, 0:1])
    dy = _b16(nd[:, :, 1:2])
    dz = _b16(nd[:, :, 2:3])
    s0 = _b16(s_ref[0:1, :][None])
    s1 = _b16(s_ref[1:2, :][None])
    s2 = _b16(s_ref[2:3, :][None])
    return jax.nn.relu(dx * s0 + dy * s1 + dz * s2)


def _conv_body(nd_ref, fs_ref, fc_ref, s_ref, o_ref, *, o):
    th = _theta(nd_ref[0], s_ref)
    act = jnp.max(th * fs_ref[0][:, :, :o], axis=1)
    o_ref[0] = fc_ref[0] + act


def _surf_body(nd_ref, s_ref, o_ref):
    th = _theta(nd_ref[0], s_ref)
    o_ref[0] = jnp.max(th, axis=1)


def _pick_bv(v, n, o):
    budget = 1 << 21
    npad = ((n + 7) // 8) * 8
    bv = budget // (npad * max(o, 128) * 4)
    bv = max(8, 1 << max(0, int(np.log2(max(bv, 1)))))
    return min(v, bv)


def _conv_act(ndn, fs, fc, sdn8):
    bs, v, n, cp = fs.shape
    o = fc.shape[2]
    BV = _pick_bv(v, n, cp)
    return pl.pallas_call(
        functools.partial(_conv_body, o=o),
        grid=(bs, v // BV),
        in_specs=[
            pl.BlockSpec((1, BV, n, 16), lambda b, i: (b, i, 0, 0)),
            pl.BlockSpec((1, BV, n, cp), lambda b, i: (b, i, 0, 0)),
            pl.BlockSpec((1, BV, o), lambda b, i: (b, i, 0)),
            pl.BlockSpec((8, o), lambda b, i: (0, 0)),
        ],
        out_specs=pl.BlockSpec((1, BV, o), lambda b, i: (b, i, 0)),
        out_shape=jax.ShapeDtypeStruct((bs, v, o), jnp.float32),
    )(ndn, fs, fc, sdn8)


def _surf_act(ndn, sdn8):
    bs, v, n, _ = ndn.shape
    o = sdn8.shape[1]
    BV = _pick_bv(v, n, max(o, 128))
    return pl.pallas_call(
        _surf_body,
        grid=(bs, v // BV),
        in_specs=[
            pl.BlockSpec((1, BV, n, 16), lambda b, i: (b, i, 0, 0)),
            pl.BlockSpec((8, o), lambda b, i: (0, 0)),
        ],
        out_specs=pl.BlockSpec((1, BV, o), lambda b, i: (b, i, 0)),
        out_shape=jax.ShapeDtypeStruct((bs, v, o), jnp.float32),
    )(ndn, sdn8)


def _mm_body(x_ref, w_ref, b_ref, o_ref, *, act):
    x = x_ref[...]
    y = jnp.dot(x, w_ref[...], preferred_element_type=jnp.float32) + b_ref[0:1, :]
    if act == "relu":
        y = jax.nn.relu(y)
    elif act == "logsoftmax":
        mx = jnp.max(y, axis=1, keepdims=True)
        e = jnp.exp(y - mx)
        y = y - mx - jnp.log(jnp.sum(e, axis=1, keepdims=True))
    o_ref[...] = y


def _mm_affine_body(x_ref, sc_ref, sh_ref, w_ref, b_ref, o_ref, *, act):
    x = jax.nn.relu(x_ref[...] * sc_ref[0:1, :] + sh_ref[0:1, :])
    y = jnp.dot(x, w_ref[...], preferred_element_type=jnp.float32) + b_ref[0:1, :]
    if act == "relu":
        y = jax.nn.relu(y)
    o_ref[...] = y


def _matmul(x, w, b, act="none", affine=None):
    M, K = x.shape
    N = w.shape[1]
    BM = min(M, 256)
    b2 = b.reshape(1, N)
    if affine is None:
        return pl.pallas_call(
            functools.partial(_mm_body, act=act),
            grid=(M // BM,),
            in_specs=[
                pl.BlockSpec((BM, K), lambda i: (i, 0)),
                pl.BlockSpec((K, N), lambda i: (0, 0)),
                pl.BlockSpec((1, N), lambda i: (0, 0)),
            ],
            out_specs=pl.BlockSpec((BM, N), lambda i: (i, 0)),
            out_shape=jax.ShapeDtypeStruct((M, N), jnp.float32),
        )(x, w, b2)
    scale, shift = affine
    return pl.pallas_call(
        functools.partial(_mm_affine_body, act=act),
        grid=(M // BM,),
        in_specs=[
            pl.BlockSpec((BM, K), lambda i: (i, 0)),
            pl.BlockSpec((1, K), lambda i: (0, 0)),
            pl.BlockSpec((1, K), lambda i: (0, 0)),
            pl.BlockSpec((K, N), lambda i: (0, 0)),
            pl.BlockSpec((1, N), lambda i: (0, 0)),
        ],
        out_specs=pl.BlockSpec((BM, N), lambda i: (i, 0)),
        out_shape=jax.ShapeDtypeStruct((M, N), jnp.float32),
    )(x, scale.reshape(1, K), shift.reshape(1, K), w, b2)


def _norm(x, axis):
    n = jnp.linalg.norm(x, axis=axis, keepdims=True)
    return x / jnp.maximum(n, 1e-12)


def _sdn8(directions):
    sdn = _norm(directions, 0)
    return jnp.concatenate([sdn, jnp.zeros((5, sdn.shape[1]), jnp.float32)], axis=0)


def _bn_affine(p, x):
    m = jnp.mean(x, axis=(0, 1))
    var = jnp.var(x, axis=(0, 1))
    inv = p["gamma"] / jnp.sqrt(var + 1e-5)
    return inv, p["beta"] - m * inv


def _bn_relu(p, x):
    sc, sh = _bn_affine(p, x)
    return jax.nn.relu(x * sc + sh)


def _pad_n(a, n):
    npad = ((n + 7) // 8) * 8
    a = a[:, :, :n]
    if npad == n:
        return a
    return jnp.concatenate([a, a[:, :, : npad - n]], axis=2)


def _conv_layer(p, ndn_n, idx_n, fm_sc_sh, dim_in, out_ch, vinfo):
    bs, v = vinfo
    fm, sc, sh = fm_sc_sh
    aff = None if sc is None else (sc, sh)
    fout = _matmul(fm.reshape(bs * v, dim_in), p["weights"], p["bias"], affine=aff)
    fout = fout.reshape(bs, v, 2 * out_ch)
    fc = fout[:, :, :out_ch]
    fs_tab = fout[:, :, out_ch:]
    fs = _gather_nd(fs_tab, idx_n)
    return _conv_act(ndn_n, fs, fc, _sdn8(p["directions"]))


def _down(p, parts, affines, dim3, vinfo):
    bs, v = vinfo
    xcat = jnp.concatenate(parts, axis=2).reshape(bs * v, dim3)
    sc = jnp.concatenate([a[0] for a in affines])
    sh = jnp.concatenate([a[1] for a in affines])
    out = _matmul(xcat, p["down_w"], p["down_b"], act="relu", affine=(sc, sh))
    return out.reshape(bs, v, -1)


def _fusion_surface(p, nb, dim, vinfo):
    nd5, _ = nb[5]
    nd20, ix20 = nb[20]
    nd100, ix100 = nb[100]
    raw_l = _surf_act(nd5, _sdn8(p["conv_l"]["directions"]))
    raw_m0 = _surf_act(nd20, _sdn8(p["conv_m0"]["directions"]))
    a_m0 = _bn_affine(p["bn_m0"], raw_m0)
    raw_m1 = _conv_layer(p["conv_m1"], nd20, ix20, (raw_m0, *a_m0), dim, dim, vinfo)
    raw_g0 = _surf_act(nd100, _sdn8(p["conv_g0"]["directions"]))
    a_g0 = _bn_affine(p["bn_g0"], raw_g0)
    raw_g1 = _conv_layer(p["conv_g1"], nd100, ix100, (raw_g0, *a_g0), dim, dim, vinfo)
    a_g1 = _bn_affine(p["bn_g1"], raw_g1)
    raw_g2 = _conv_layer(p["conv_g2"], nd100, ix100, (raw_g1, *a_g1), dim, dim, vinfo)
    return _down(
        p,
        [raw_l, raw_m1, raw_g2],
        [_bn_affine(p["bn_l"], raw_l), _bn_affine(p["bn_m1"], raw_m1), _bn_affine(p["bn_g2"], raw_g2)],
        dim * 3,
        vinfo,
    )


def _fusion(p, nb, feat, dim_in, dim, vinfo):
    nd5, ix5 = nb[5]
    nd20, ix20 = nb[20]
    nd100, ix100 = nb[100]
    raw_l = _conv_layer(p["conv_l"], nd5, ix5, (feat, None, None), dim_in, dim, vinfo)
    raw_m0 = _conv_layer(p["conv_m0"], nd20, ix20, (feat, None, None), dim_in, dim, vinfo)
    a_m0 = _bn_affine(p["bn_m0"], raw_m0)
    raw_m1 = _conv_layer(p["conv_m1"], nd20, ix20, (raw_m0, *a_m0), dim, dim, vinfo)
    raw_g0 = _conv_layer(p["conv_g0"], nd100, ix100, (feat, None, None), dim_in, dim, vinfo)
    a_g0 = _bn_affine(p["bn_g0"], raw_g0)
    raw_g1 = _conv_layer(p["conv_g1"], nd100, ix100, (raw_g0, *a_g0), dim, dim, vinfo)
    a_g1 = _bn_affine(p["bn_g1"], raw_g1)
    raw_g2 = _conv_layer(p["conv_g0"], nd100, ix100, (raw_g1, *a_g1), dim, dim, vinfo)
    return _down(
        p,
        [raw_l, raw_m1, raw_g2],
        [_bn_affine(p["bn_l"], raw_l), _bn_affine(p["bn_m1"], raw_m1), _bn_affine(p["bn_g0"], raw_g2)],
        dim * 3,
        vinfo,
    )


def _scale_setup(verts):
    bs, v, _ = verts.shape
    top = _topk_from_dist(_knn_dist(verts), 101)
    idx101 = top[:, :, 1:]
    v16 = jnp.pad(verts, ((0, 0), (0, 0), (0, 13)))
    nxyz = _gather_nd(v16, idx101)
    ndn = _ndn16(nxyz, v16)
    nb = {n: (_pad_n(ndn, n), _pad_n(idx101, n)) for n in (5, 20, 100)}
    return top, nb


def _pool(verts, fm, top, rate, seed):
    bs, v, _ = verts.shape
    nbr = top[:, :, 1:5]
    pooled = jnp.max(_gather_nd(fm, nbr), axis=2)
    keep = jnp.asarray(np.random.RandomState(seed).permutation(v)[: v // rate])
    return verts[:, keep, :], pooled[:, keep, :]


def kernel(vertices, onehot, params):
    verts = jnp.transpose(vertices, (0, 2, 1))
    bs, v, _ = verts.shape
    vinfo0 = (bs, v)

    top0, nb0 = _scale_setup(verts)
    fm_0 = _fusion_surface(params["conv_0"], nb0, 32, vinfo0)
    fm_1 = _fusion(params["conv_1"], nb0, fm_0, 64, 64, vinfo0)
    v1, fp1 = _pool(verts, fm_1, top0, 4, 1)

    vinfo1 = (bs, v // 4)
    top1, nb1 = _scale_setup(v1)
    fm_2 = _fusion(params["conv_2"], nb1, fp1, 128, 128, vinfo1)
    fm_3 = _fusion(params["conv_3"], nb1, fm_2, 256, 256, vinfo1)
    v2, fp2 = _pool(v1, fm_3, top1, 4, 2)

    vinfo2 = (bs, v // 16)
    top2, nb2 = _scale_setup(v2)
    fm_4 = _fusion(params["conv_4"], nb2, fp2, 512, 512, vinfo2)
    f_global = jnp.max(fm_4, axis=1)

    n1 = _topk_from_dist(_nearest_dist(verts, v1), 1)[:, :, 0]
    n2 = _topk_from_dist(_nearest_dist(verts, v2), 1)[:, :, 0]
    fm_2u = _gather_nd(fm_2, n1)
    fm_3u = _gather_nd(fm_3, n1)
    fm_4u = _gather_nd(fm_4, n2)

    fg = jnp.broadcast_to(f_global[:, None, :], (bs, v, f_global.shape[-1]))
    oh = jnp.broadcast_to(onehot[:, None, :], (bs, v, onehot.shape[-1]))
    fuse = jnp.concatenate([fm_0, fm_1, fm_2u, fm_3u, fm_4u, fg, oh], axis=2)
    dimf = fuse.shape[-1]
    x = _matmul(fuse.reshape(bs * v, dimf), params["c1_w"], params["c1_b"], act="relu")
    x = _matmul(x, params["c2_w"], params["c2_b"], act="relu")
    x = _matmul(x, params["c3_w"], params["c3_b"], act="logsoftmax")
    return x.reshape(bs, v, -1)
